# Initial kernel scaffold; baseline (speedup 1.0000x reference)
import jax
import jax.numpy as jnp
from jax.experimental import pallas as pl


def kernel(tag1, tag2, tag3, ind1, ind2, ind3, mask):
    raise NotImplementedError("write your pallas kernel here")


# speedup vs baseline: 1.0402x; 1.0402x over previous
import functools

import jax
import jax.numpy as jnp
from jax import lax
from jax.experimental import pallas as pl
from jax.experimental.pallas import tpu as pltpu
from jax.experimental.pallas import tpu_sc as plsc

B, K, H, W = 32, 256, 512, 512
HW = H * W
NC, NS = 2, 16
KW = 128
KR = K // KW
EPS = 1e-4

@functools.cache
def _make_sc_gather():
    mesh = plsc.VectorSubcoreMesh(
        core_axis_name="c", subcore_axis_name="s",
        num_cores=NC, num_subcores=NS)

    @functools.partial(
        pl.kernel,
        out_type=jax.ShapeDtypeStruct((3, B, KR, KW), jnp.float32),
        mesh=mesh,
        scratch_types=[
            pltpu.VMEM((KR, KW), jnp.int32),
            pltpu.VMEM((KR, KW), jnp.int32),
            pltpu.VMEM((KR, KW), jnp.float32),
            pltpu.SemaphoreType.DMA,
        ],
    )
    def _sc_gather(t1, t2, t3, i1, i2, i3, out, idx_v, adj_v, val_v, sem):
        b = lax.axis_index("s") * NC + lax.axis_index("c")
        off = b * HW
        for i, (t, ind) in enumerate(((t1, i1), (t2, i2), (t3, i3))):
            pltpu.sync_copy(ind.at[b], idx_v)
            for r in range(KR):
                for l in range(KW // 16):
                    sl = (r, pl.ds(l * 16, 16))
                    adj_v[sl] = idx_v[sl] + off
            copies = [
                pltpu.async_copy(t.at[adj_v.at[r]], val_v.at[r], sem)
                for r in range(KR)
            ]
            for c in copies:
                c.wait()
            pltpu.sync_copy(val_v, out.at[i, b])

    return _sc_gather


BB = 8


_OUTER = (((0,), (0,)), ((), ()))


def _loss_body(t0, t1, t2, mk, pull_ref, push_ref):
    g = pl.program_id(0)
    third = jnp.float32(1.0 / 3.0)
    r0, r1, r2 = t0[...], t1[...], t2[...]
    mean8 = (r0 + r1 + r2) * third
    km8 = mk[...]
    sq8 = (jnp.square(r0 - mean8) + jnp.square(r1 - mean8)
           + jnp.square(r2 - mean8)) * km8
    ones_row = jnp.ones((1, K), jnp.float32)

    pull_acc = jnp.float32(0.0)
    push_acc = jnp.float32(0.0)
    for i in range(BB):
        km = km8[i:i + 1, :]
        num = jnp.sum(km)
        inv_num = 1.0 / (num + EPS)
        pull_acc += jnp.sum(sq8[i:i + 1, :]) * inv_num

        num2 = (num - 1.0) * num
        mrow = mean8[i:i + 1, :]
        mcol = lax.dot_general(mrow, ones_row, _OUTER,
                               preferred_element_type=jnp.float32)
        pair = lax.dot_general(km, km, _OUTER,
                               preferred_element_type=jnp.float32)
        dist = jnp.maximum(1.0 - jnp.abs(mrow - mcol), 0.0)
        dist = (dist - inv_num) / (num2 + EPS)
        push_acc += jnp.sum(dist * pair)

    @pl.when(g == 0)
    def _init():
        pull_ref[0, 0] = jnp.float32(0.0)
        push_ref[0, 0] = jnp.float32(0.0)

    pull_ref[0, 0] += pull_acc
    push_ref[0, 0] += push_acc


def _make_loss_call():
    return pl.pallas_call(
        _loss_body,
        grid=(B // BB,),
        in_specs=[
            pl.BlockSpec((BB, K), lambda g: (g, 0)),
            pl.BlockSpec((BB, K), lambda g: (g, 0)),
            pl.BlockSpec((BB, K), lambda g: (g, 0)),
            pl.BlockSpec((BB, K), lambda g: (g, 0)),
        ],
        out_specs=[
            pl.BlockSpec((1, 1), lambda g: (0, 0),
                         memory_space=pltpu.MemorySpace.SMEM),
            pl.BlockSpec((1, 1), lambda g: (0, 0),
                         memory_space=pltpu.MemorySpace.SMEM),
        ],
        out_shape=[
            jax.ShapeDtypeStruct((1, 1), jnp.float32),
            jax.ShapeDtypeStruct((1, 1), jnp.float32),
        ],
    )


_loss_call = _make_loss_call()


def kernel(tag1, tag2, tag3, ind1, ind2, ind3, mask):
    gathered = _make_sc_gather()(
        tag1.reshape(B * HW),
        tag2.reshape(B * HW),
        tag3.reshape(B * HW),
        ind1.astype(jnp.int32).reshape(B, KR, KW),
        ind2.astype(jnp.int32).reshape(B, KR, KW),
        ind3.astype(jnp.int32).reshape(B, KR, KW),
    )
    t = gathered.reshape(3, B, K)
    t0, t1, t2 = t[0], t[1], t[2]
    maskf = mask.astype(jnp.float32)
    pull, push = _loss_call(t0, t1, t2, maskf)
    return pull[0, 0], push[0, 0]


# speedup vs baseline: 3.1803x; 3.0572x over previous
import functools

import jax
import jax.numpy as jnp
from jax import lax
from jax.experimental import pallas as pl
from jax.experimental.pallas import tpu as pltpu
from jax.experimental.pallas import tpu_sc as plsc

B, K, H, W = 32, 256, 512, 512
HW = H * W
NC, NS = 2, 16
KW = 128
KR = K // KW
EPS = 1e-4

@functools.cache
def _make_sc_gather():
    mesh = plsc.VectorSubcoreMesh(
        core_axis_name="c", subcore_axis_name="s",
        num_cores=NC, num_subcores=NS)

    @functools.partial(
        pl.kernel,
        out_type=jax.ShapeDtypeStruct((3, B, KR, KW), jnp.float32),
        mesh=mesh,
        scratch_types=[
            pltpu.VMEM((KR, KW), jnp.int32),
            pltpu.VMEM((KR, KW), jnp.int32),
            pltpu.VMEM((KR, KW), jnp.float32),
            pltpu.SemaphoreType.DMA,
        ],
    )
    def _sc_gather(t1, t2, t3, i1, i2, i3, out, idx_v, adj_v, val_v, sem):
        b = lax.axis_index("s") * NC + lax.axis_index("c")
        base = b * HW
        for i, (t, ind) in enumerate(((t1, i1), (t2, i2), (t3, i3))):
            pltpu.sync_copy(ind.at[b], idx_v)
            for r in range(KR):
                for l in range(KW // 16):
                    sl = (r, pl.ds(l * 16, 16))
                    v = idx_v[sl]
                    adj_v[sl] = (
                        (v & 0x3F000)
                        | ((v & 0x180) << 3)
                        | ((v & 0xE00) >> 2)
                        | (v & 0x7F)
                    ) + base
            copies = [
                pltpu.async_copy(t.at[adj_v.at[r]], val_v.at[r], sem)
                for r in range(KR)
            ]
            for c in copies:
                c.wait()
            pltpu.sync_copy(val_v, out.at[i, b])

    return _sc_gather


BB = 8


_OUTER = (((0,), (0,)), ((), ()))


def _loss_body(t0, t1, t2, mk, pull_ref, push_ref):
    g = pl.program_id(0)
    third = jnp.float32(1.0 / 3.0)
    r0, r1, r2 = t0[...], t1[...], t2[...]
    mean8 = (r0 + r1 + r2) * third
    km8 = mk[...]
    sq8 = (jnp.square(r0 - mean8) + jnp.square(r1 - mean8)
           + jnp.square(r2 - mean8)) * km8
    ones_row = jnp.ones((1, K), jnp.float32)

    pull_acc = jnp.float32(0.0)
    push_acc = jnp.float32(0.0)
    for i in range(BB):
        km = km8[i:i + 1, :]
        num = jnp.sum(km)
        inv_num = 1.0 / (num + EPS)
        pull_acc += jnp.sum(sq8[i:i + 1, :]) * inv_num

        num2 = (num - 1.0) * num
        mrow = mean8[i:i + 1, :]
        mcol = lax.dot_general(mrow, ones_row, _OUTER,
                               preferred_element_type=jnp.float32)
        pair = lax.dot_general(km, km, _OUTER,
                               preferred_element_type=jnp.float32)
        dist = jnp.maximum(1.0 - jnp.abs(mrow - mcol), 0.0)
        dist = (dist - inv_num) / (num2 + EPS)
        push_acc += jnp.sum(dist * pair)

    @pl.when(g == 0)
    def _init():
        pull_ref[0, 0] = jnp.float32(0.0)
        push_ref[0, 0] = jnp.float32(0.0)

    pull_ref[0, 0] += pull_acc
    push_ref[0, 0] += push_acc


def _make_loss_call():
    return pl.pallas_call(
        _loss_body,
        grid=(B // BB,),
        in_specs=[
            pl.BlockSpec((BB, K), lambda g: (g, 0)),
            pl.BlockSpec((BB, K), lambda g: (g, 0)),
            pl.BlockSpec((BB, K), lambda g: (g, 0)),
            pl.BlockSpec((BB, K), lambda g: (g, 0)),
        ],
        out_specs=[
            pl.BlockSpec((1, 1), lambda g: (0, 0),
                         memory_space=pltpu.MemorySpace.SMEM),
            pl.BlockSpec((1, 1), lambda g: (0, 0),
                         memory_space=pltpu.MemorySpace.SMEM),
        ],
        out_shape=[
            jax.ShapeDtypeStruct((1, 1), jnp.float32),
            jax.ShapeDtypeStruct((1, 1), jnp.float32),
        ],
    )


_loss_call = _make_loss_call()


def _tile_view(tag):
    v = tag.reshape(B, H // 8, 8, W // 128, 128)
    v = v.transpose(0, 1, 3, 2, 4)
    return v.reshape(B * HW)


def kernel(tag1, tag2, tag3, ind1, ind2, ind3, mask):
    gathered = _make_sc_gather()(
        _tile_view(tag1),
        _tile_view(tag2),
        _tile_view(tag3),
        ind1.astype(jnp.int32).reshape(B, KR, KW),
        ind2.astype(jnp.int32).reshape(B, KR, KW),
        ind3.astype(jnp.int32).reshape(B, KR, KW),
    )
    t = gathered.reshape(3, B, K)
    t0, t1, t2 = t[0], t[1], t[2]
    maskf = mask.astype(jnp.float32)
    pull, push = _loss_call(t0, t1, t2, maskf)
    return pull[0, 0], push[0, 0]
